# baseline (speedup 1.0000x reference)
import jax
import jax.numpy as jnp
from jax import lax
from jax.experimental import pallas as pl
from jax.experimental.pallas import tpu as pltpu
from jax.experimental.pallas import tpu_sc as plsc

B, T, D, V = 4, 2048, 768, 100000
NC, NS = 2, 16
NW = NC * NS
TPW = T // NW
PH = 16
NPH = TPW // PH
ROWS = B * PH
LANES = 16


def _emb_body(idx_hbm, vocab_hbm, pos_hbm, out_hbm,
              idx_v, p0, p1, buf0, buf1,
              sem_i, sem_p0, sem_p1, sg0, sg1, ss0, ss1):
    pos_bufs = (p0, p1)
    bufs = (buf0, buf1)
    psems = (sem_p0, sem_p1)
    gsems = (sg0, sg1)
    ssems = (ss0, ss1)

    wid = lax.axis_index("s") * NC + lax.axis_index("c")
    t0 = wid * TPW

    cp_idx = [pltpu.async_copy(
                  idx_hbm.at[pl.ds(b * T + t0 + q * PH, PH)],
                  idx_v.at[pl.ds(q * ROWS + b * PH, PH)], sem_i)
              for q in range(NPH) for b in range(B)]

    def start_pos(q):
        return pltpu.async_copy(pos_hbm.at[pl.ds(t0 + q * PH, PH)],
                                pos_bufs[q % 2], psems[q % 2])

    def start_gather(q):
        return pltpu.async_copy(
            vocab_hbm.at[idx_v.at[pl.ds(q * ROWS, ROWS)]],
            bufs[q % 2], gsems[q % 2])

    for cp in cp_idx:
        cp.wait()
    pos_cps = {0: start_pos(0), 1: start_pos(1)}
    gather_cps = {0: start_gather(0), 1: start_gather(1)}
    store_cps = {}

    for q in range(NPH):
        g = q % 2
        pos_cps[q].wait()
        gather_cps[q].wait()
        pos_b = pos_bufs[g]
        buf = bufs[g]

        @plsc.parallel_loop(0, PH, 1, unroll=2)
        def _(r):
            for c in range(0, D, LANES):
                cs = pl.ds(c, LANES)
                pv = pos_b[r, cs]
                for b in range(B):
                    buf[b * PH + r, cs] = buf[b * PH + r, cs] + pv

        store_cps[q] = [pltpu.async_copy(
                            buf.at[pl.ds(b * PH, PH)],
                            out_hbm.at[b, pl.ds(t0 + q * PH, PH)],
                            ssems[g])
                        for b in range(B)]

        if q + 2 < NPH:
            pos_cps[q + 2] = start_pos(q + 2)
            for cp in store_cps[q]:
                cp.wait()
            gather_cps[q + 2] = start_gather(q + 2)

    for q in (NPH - 2, NPH - 1):
        for cp in store_cps[q]:
            cp.wait()


def kernel(idx, pos, vocab_table, pos_table):
    del pos
    idx = idx.astype(jnp.int32).reshape(B * T)
    mesh = plsc.VectorSubcoreMesh(core_axis_name="c", subcore_axis_name="s",
                                  num_cores=NC, num_subcores=NS)
    emb = pl.kernel(
        _emb_body,
        out_type=jax.ShapeDtypeStruct((B, T, D), jnp.float32),
        mesh=mesh,
        scratch_types=[
            pltpu.VMEM((NPH * ROWS,), jnp.int32),
            pltpu.VMEM((PH, D), jnp.float32),
            pltpu.VMEM((PH, D), jnp.float32),
            pltpu.VMEM((ROWS, D), jnp.float32),
            pltpu.VMEM((ROWS, D), jnp.float32),
        ] + [pltpu.SemaphoreType.DMA] * 7,
    )
    return emb(idx, vocab_table, pos_table)


# speedup vs baseline: 1.8012x; 1.8012x over previous
import jax
import jax.numpy as jnp
from jax import lax
from jax.experimental import pallas as pl
from jax.experimental.pallas import tpu as pltpu
from jax.experimental.pallas import tpu_sc as plsc

B, T, D, V = 4, 2048, 768, 100000
NC, NS = 2, 16
NW = NC * NS
TPW = T // NW
PH = 16
NPH = TPW // PH
LANES = 16


def _emb_body(idx_hbm, vocab_hbm, pos_hbm, out_hbm,
              idx_v, p0, p1, b00, b01, b02, b03, b10, b11, b12, b13,
              sem_i, sem_p0, sem_p1, sg0, sg1, ss0, ss1):
    pos_bufs = (p0, p1)
    bufs = ((b00, b01, b02, b03), (b10, b11, b12, b13))
    psems = (sem_p0, sem_p1)
    gsems = (sg0, sg1)
    ssems = (ss0, ss1)

    wid = lax.axis_index("s") * NC + lax.axis_index("c")
    t0 = wid * TPW

    cp_idx = [pltpu.async_copy(idx_hbm.at[pl.ds(b * T + t0, TPW)],
                               idx_v.at[pl.ds(b * TPW, TPW)], sem_i)
              for b in range(B)]

    def start_phase(q):
        g = q % 2
        pcp = pltpu.async_copy(pos_hbm.at[pl.ds(t0 + q * PH, PH)],
                               pos_bufs[g], psems[g])
        gcps = [pltpu.async_copy(
                    vocab_hbm.at[idx_v.at[pl.ds(b * TPW + q * PH, PH)]],
                    bufs[g][b], gsems[g])
                for b in range(B)]
        return [pcp] + gcps

    for cp in cp_idx:
        cp.wait()
    phases = {0: start_phase(0), 1: start_phase(1)}
    stores = {}

    for q in range(NPH):
        g = q % 2
        for cp in phases[q]:
            cp.wait()
        pos_b = pos_bufs[g]
        grp = bufs[g]

        @plsc.parallel_loop(0, PH, 1, unroll=2)
        def _(r):
            for c in range(0, D, LANES):
                cs = pl.ds(c, LANES)
                pv = pos_b[r, cs]
                for b in range(B):
                    grp[b][r, cs] = grp[b][r, cs] + pv

        stores[q] = [pltpu.async_copy(
                         grp[b], out_hbm.at[b, pl.ds(t0 + q * PH, PH)],
                         ssems[g])
                     for b in range(B)]
        if q + 2 <= NPH - 1:
            for cp in stores[q]:
                cp.wait()
            phases[q + 2] = start_phase(q + 2)

    for q in (NPH - 2, NPH - 1):
        for cp in stores[q]:
            cp.wait()


def kernel(idx, pos, vocab_table, pos_table):
    del pos
    idx = idx.astype(jnp.int32).reshape(B * T)
    mesh = plsc.VectorSubcoreMesh(core_axis_name="c", subcore_axis_name="s",
                                  num_cores=NC, num_subcores=NS)
    emb = pl.kernel(
        _emb_body,
        out_type=jax.ShapeDtypeStruct((B, T, D), jnp.float32),
        mesh=mesh,
        scratch_types=[
            pltpu.VMEM((B * TPW,), jnp.int32),
            pltpu.VMEM((PH, D), jnp.float32),
            pltpu.VMEM((PH, D), jnp.float32),
        ] + [pltpu.VMEM((PH, D), jnp.float32) for _ in range(2 * B)]
          + [pltpu.SemaphoreType.DMA] * 7,
    )
    return emb(idx, vocab_table, pos_table)


# speedup vs baseline: 1.8057x; 1.0025x over previous
import jax
import jax.numpy as jnp
from jax import lax
from jax.experimental import pallas as pl
from jax.experimental.pallas import tpu as pltpu
from jax.experimental.pallas import tpu_sc as plsc

B, T, D, V = 4, 2048, 768, 100000
NC, NS = 2, 16
NW = NC * NS
TPW = T // NW
PH = 16
NPH = TPW // PH
LANES = 16


def _emb_body(idx_hbm, vocab_hbm, pos_hbm, out_hbm,
              idx_v, p0, p1, b00, b01, b02, b03, b10, b11, b12, b13,
              sem_i, sem_p0, sem_p1, sg0, sg1, ss0, ss1):
    pos_bufs = (p0, p1)
    bufs = ((b00, b01, b02, b03), (b10, b11, b12, b13))
    psems = (sem_p0, sem_p1)
    gsems = (sg0, sg1)
    ssems = (ss0, ss1)

    wid = lax.axis_index("c") * NS + lax.axis_index("s")
    t0 = wid * TPW

    cp_idx = [pltpu.async_copy(idx_hbm.at[pl.ds(b * T + t0, TPW)],
                               idx_v.at[pl.ds(b * TPW, TPW)], sem_i)
              for b in range(B)]

    def start_phase(q):
        g = q % 2
        pcp = pltpu.async_copy(pos_hbm.at[pl.ds(t0 + q * PH, PH)],
                               pos_bufs[g], psems[g])
        gcps = [pltpu.async_copy(
                    vocab_hbm.at[idx_v.at[pl.ds(b * TPW + q * PH, PH)]],
                    bufs[g][b], gsems[g])
                for b in range(B)]
        return [pcp] + gcps

    for cp in cp_idx:
        cp.wait()
    phases = {0: start_phase(0), 1: start_phase(1)}
    stores = {}

    for q in range(NPH):
        g = q % 2
        for cp in phases[q]:
            cp.wait()
        pos_b = pos_bufs[g]
        grp = bufs[g]

        @plsc.parallel_loop(0, PH, 1, unroll=2)
        def _(r):
            for c in range(0, D, LANES):
                cs = pl.ds(c, LANES)
                pv = pos_b[r, cs]
                for b in range(B):
                    grp[b][r, cs] = grp[b][r, cs] + pv

        stores[q] = [pltpu.async_copy(
                         grp[b], out_hbm.at[b, pl.ds(t0 + q * PH, PH)],
                         ssems[g])
                     for b in range(B)]
        if q + 2 <= NPH - 1:
            for cp in stores[q]:
                cp.wait()
            phases[q + 2] = start_phase(q + 2)

    for q in (NPH - 2, NPH - 1):
        for cp in stores[q]:
            cp.wait()


def kernel(idx, pos, vocab_table, pos_table):
    del pos
    idx = idx.astype(jnp.int32).reshape(B * T)
    mesh = plsc.VectorSubcoreMesh(core_axis_name="c", subcore_axis_name="s",
                                  num_cores=NC, num_subcores=NS)
    emb = pl.kernel(
        _emb_body,
        out_type=jax.ShapeDtypeStruct((B, T, D), jnp.float32),
        mesh=mesh,
        scratch_types=[
            pltpu.VMEM((B * TPW,), jnp.int32),
            pltpu.VMEM((PH, D), jnp.float32),
            pltpu.VMEM((PH, D), jnp.float32),
        ] + [pltpu.VMEM((PH, D), jnp.float32) for _ in range(2 * B)]
          + [pltpu.SemaphoreType.DMA] * 7,
    )
    return emb(idx, vocab_table, pos_table)
